# baseline (speedup 1.0000x reference)
import functools

import jax
import jax.numpy as jnp
from jax import lax
from jax.experimental import pallas as pl
from jax.experimental.pallas import tpu as pltpu
from jax.experimental.pallas import tpu_sc as plsc

VOCAB = 100000
D = 128
BATCH = 4096
SEQ = 200
N = BATCH * SEQ
EPS = 1e-12

NC = 2
NS = 16
NW = NC * NS
PER_W = N // NW
C = 128
NCHUNK = PER_W // C
NBUF = 4
L = 16


def _rsqrt_nr(x):
    i = lax.bitcast_convert_type(x, jnp.int32)
    i = 0x5F3759DF - lax.shift_right_logical(i, 1)
    y = lax.bitcast_convert_type(i, jnp.float32)
    for _ in range(3):
        y = y * (1.5 - 0.5 * x * y * y)
    return y


def _ln_chunk(buf, a_scr, b_scr, g_v, b_v):
    iota = lax.iota(jnp.int32, L)
    zeros = jnp.zeros((L,), jnp.float32)

    for grp in range(C // L):
        rows = iota + (grp * L)

        @pl.loop(0, D, init_carry=(zeros, zeros), unroll=8)
        def _acc(d, carry, rows=rows):
            s, q = carry
            x = plsc.load_gather(buf, [rows, jnp.full((L,), d, jnp.int32)])
            return (s + x, q + x * x)

        s, q = _acc
        mean = s * (1.0 / D)
        var = jnp.maximum(q * (1.0 / D) - mean * mean, 0.0) + EPS
        rstd = _rsqrt_nr(var)
        a_scr[pl.ds(grp * L, L)] = rstd
        b_scr[pl.ds(grp * L, L)] = -mean * rstd

    gs = [g_v[pl.ds(k * L, L)] for k in range(D // L)]
    bs = [b_v[pl.ds(k * L, L)] for k in range(D // L)]

    @pl.loop(0, C)
    def _norm(j):
        aj = plsc.load_gather(a_scr, [jnp.full((L,), j, jnp.int32)])
        bj = plsc.load_gather(b_scr, [jnp.full((L,), j, jnp.int32)])
        for k in range(D // L):
            x = buf[j, pl.ds(k * L, L)]
            buf[j, pl.ds(k * L, L)] = (x * aj + bj) * gs[k] + bs[k]


def _sc_body(ids_hbm, table_hbm, gamma_hbm, beta_hbm, out_hbm,
             idx_all, bufs, a_scr, b_scr, g_v, b_v, sems_in, sems_out):
    wid = lax.axis_index("s") * NC + lax.axis_index("c")
    base = wid * PER_W

    pltpu.sync_copy(ids_hbm.at[pl.ds(base, PER_W)], idx_all)
    pltpu.sync_copy(gamma_hbm, g_v)
    pltpu.sync_copy(beta_hbm, b_v)

    def fire_gather(g, s):
        pltpu.async_copy(
            table_hbm.at[idx_all.at[pl.ds(g * C, C)]], bufs[s], sems_in[s])

    def wait_gather(g, s):
        pltpu.make_async_copy(
            table_hbm.at[idx_all.at[pl.ds(g * C, C)]], bufs[s], sems_in[s]).wait()

    def fire_write(g, s):
        pltpu.async_copy(bufs[s], out_hbm.at[pl.ds(base + g * C, C)], sems_out[s])

    def wait_write(g, s):
        pltpu.make_async_copy(
            bufs[s], out_hbm.at[pl.ds(base + g * C, C)], sems_out[s]).wait()

    def step(g, s, s2, first, last):
        wait_gather(g, s)
        if not first:
            wait_write(g - 2, s2)
        if not last:
            fire_gather(g + 2, s2)
        _ln_chunk(bufs[s], a_scr, b_scr, g_v, b_v)
        fire_write(g, s)

    fire_gather(0, 0)
    fire_gather(1, 1)
    for g in (0, 1):
        step(g, g % NBUF, (g + 2) % NBUF, first=True, last=False)

    @pl.loop(2, NCHUNK - 2, step=NBUF)
    def _outer(g0):
        for b in range(NBUF):
            s = (2 + b) % NBUF
            step(g0 + b, s, (s + 2) % NBUF, first=False, last=False)

    for g in (NCHUNK - 2, NCHUNK - 1):
        step(g, g % NBUF, (g - 2) % NBUF, first=False, last=True)
    for g in (NCHUNK - 2, NCHUNK - 1):
        wait_write(g, g % NBUF)


_sc_fused = functools.partial(
    pl.kernel,
    out_type=jax.ShapeDtypeStruct((N, D), jnp.float32),
    mesh=plsc.VectorSubcoreMesh(core_axis_name="c", subcore_axis_name="s"),
    compiler_params=pltpu.CompilerParams(needs_layout_passes=False),
    scratch_types=[
        pltpu.VMEM((PER_W,), jnp.int32),
        [pltpu.VMEM((C, D), jnp.float32) for _ in range(NBUF)],
        pltpu.VMEM((C,), jnp.float32),
        pltpu.VMEM((C,), jnp.float32),
        pltpu.VMEM((D,), jnp.float32),
        pltpu.VMEM((D,), jnp.float32),
        [pltpu.SemaphoreType.DMA for _ in range(NBUF)],
        [pltpu.SemaphoreType.DMA for _ in range(NBUF)],
    ],
)(_sc_body)


def kernel(input_ids, table, gamma, beta):
    ids = input_ids.reshape(-1).astype(jnp.int32)
    out = _sc_fused(ids, table, gamma, beta)
    return out.reshape(BATCH, SEQ, D)


# speedup vs baseline: 2.7130x; 2.7130x over previous
import functools

import jax
import jax.numpy as jnp
from jax import lax
from jax.experimental import pallas as pl
from jax.experimental.pallas import tpu as pltpu
from jax.experimental.pallas import tpu_sc as plsc

VOCAB = 100000
D = 128
BATCH = 4096
SEQ = 200
N = BATCH * SEQ
EPS = 1e-12

NC = 2
NS = 16
NW = NC * NS
PER_W = N // NW
C = 128
NCHUNK = PER_W // C


NBUF = 4


def _sc_gather_body(ids_hbm, table_hbm, out_hbm, idx_all, bufs, sems_in, sems_out):
    wid = lax.axis_index("s") * NC + lax.axis_index("c")
    base = wid * PER_W

    pltpu.sync_copy(ids_hbm.at[pl.ds(base, PER_W)], idx_all)

    def fire_gather(g, s):
        pltpu.async_copy(
            table_hbm.at[idx_all.at[pl.ds(g * C, C)]], bufs[s], sems_in[s])

    def wait_gather(g, s):
        pltpu.make_async_copy(
            table_hbm.at[idx_all.at[pl.ds(g * C, C)]], bufs[s], sems_in[s]).wait()

    def fire_write(g, s):
        pltpu.async_copy(bufs[s], out_hbm.at[pl.ds(base + g * C, C)], sems_out[s])

    def wait_write(g, s):
        pltpu.make_async_copy(
            bufs[s], out_hbm.at[pl.ds(base + g * C, C)], sems_out[s]).wait()

    fire_gather(0, 0)
    fire_gather(1, 1)
    for g in (0, 1):
        s = g % NBUF
        wait_gather(g, s)
        fire_write(g, s)
        fire_gather(g + 2, (g + 2) % NBUF)

    @pl.loop(2, NCHUNK - 2, step=NBUF)
    def _outer(g0):
        for b in range(NBUF):
            g = g0 + b
            s = (2 + b) % NBUF
            wait_gather(g, s)
            fire_write(g, s)
            wait_write(g - 2, (s + 2) % NBUF)
            fire_gather(g + 2, (s + 2) % NBUF)

    for g in (NCHUNK - 2, NCHUNK - 1):
        s = g % NBUF
        wait_gather(g, s)
        fire_write(g, s)
        wait_write(g - 2, (g - 2) % NBUF)
    for g in (NCHUNK - 2, NCHUNK - 1):
        wait_write(g, g % NBUF)


_sc_gather = functools.partial(
    pl.kernel,
    out_type=jax.ShapeDtypeStruct((N, D), jnp.float32),
    mesh=plsc.VectorSubcoreMesh(core_axis_name="c", subcore_axis_name="s"),
    scratch_types=[
        pltpu.VMEM((PER_W,), jnp.int32),
        [pltpu.VMEM((C, D), jnp.float32) for _ in range(NBUF)],
        [pltpu.SemaphoreType.DMA for _ in range(NBUF)],
        [pltpu.SemaphoreType.DMA for _ in range(NBUF)],
    ],
)(_sc_gather_body)


def _ln_body(x_ref, g_ref, b_ref, o_ref):
    x = x_ref[...]
    mean = jnp.mean(x, axis=1, keepdims=True)
    cent = x - mean
    var = jnp.mean(cent * cent, axis=1, keepdims=True)
    o_ref[...] = cent * lax.rsqrt(var + EPS) * g_ref[...] + b_ref[...]


_LN_ROWS = 8192


def _tc_layernorm(x, gamma, beta):
    return pl.pallas_call(
        _ln_body,
        grid=(N // _LN_ROWS,),
        in_specs=[
            pl.BlockSpec((_LN_ROWS, D), lambda i: (i, 0)),
            pl.BlockSpec((1, D), lambda i: (0, 0)),
            pl.BlockSpec((1, D), lambda i: (0, 0)),
        ],
        out_specs=pl.BlockSpec((_LN_ROWS, D), lambda i: (i, 0)),
        out_shape=jax.ShapeDtypeStruct((N, D), jnp.float32),
    )(x, gamma.reshape(1, D), beta.reshape(1, D))


def kernel(input_ids, table, gamma, beta):
    ids = input_ids.reshape(-1).astype(jnp.int32)
    rows = _sc_gather(ids, table)
    out = _tc_layernorm(rows, gamma, beta)
    return out.reshape(BATCH, SEQ, D)


# speedup vs baseline: 2.8471x; 1.0494x over previous
import functools

import jax
import jax.numpy as jnp
from jax import lax
from jax.experimental import pallas as pl
from jax.experimental.pallas import tpu as pltpu
from jax.experimental.pallas import tpu_sc as plsc

VOCAB = 100000
D = 128
BATCH = 4096
SEQ = 200
N = BATCH * SEQ
EPS = 1e-12

NC = 2
NS = 16
NW = NC * NS
PER_W = N // NW
C = 128
NCHUNK = PER_W // C


NBUF = 4


def _sc_gather_body(ids_hbm, table_hbm, out_hbm, idx_all, bufs, sems_in, sems_out):
    wid = lax.axis_index("s") * NC + lax.axis_index("c")
    base = wid * PER_W

    pltpu.sync_copy(ids_hbm.at[pl.ds(base, PER_W)], idx_all)

    def fire_gather(g, s):
        pltpu.async_copy(
            table_hbm.at[idx_all.at[pl.ds(g * C, C)]], bufs[s], sems_in[s])

    def wait_gather(g, s):
        pltpu.make_async_copy(
            table_hbm.at[idx_all.at[pl.ds(g * C, C)]], bufs[s], sems_in[s]).wait()

    def fire_write(g, s):
        pltpu.async_copy(bufs[s], out_hbm.at[pl.ds(base + g * C, C)], sems_out[s])

    def wait_write(g, s):
        pltpu.make_async_copy(
            bufs[s], out_hbm.at[pl.ds(base + g * C, C)], sems_out[s]).wait()

    fire_gather(0, 0)
    fire_gather(1, 1)
    for g in (0, 1):
        s = g % NBUF
        wait_gather(g, s)
        fire_write(g, s)
        fire_gather(g + 2, (g + 2) % NBUF)

    @pl.loop(2, NCHUNK - 2, step=NBUF)
    def _outer(g0):
        for b in range(NBUF):
            g = g0 + b
            s = (2 + b) % NBUF
            wait_gather(g, s)
            fire_write(g, s)
            wait_write(g - 2, (s + 2) % NBUF)
            fire_gather(g + 2, (s + 2) % NBUF)

    for g in (NCHUNK - 2, NCHUNK - 1):
        s = g % NBUF
        wait_gather(g, s)
        fire_write(g, s)
        wait_write(g - 2, (g - 2) % NBUF)
    for g in (NCHUNK - 2, NCHUNK - 1):
        wait_write(g, g % NBUF)


_sc_gather = functools.partial(
    pl.kernel,
    out_type=jax.ShapeDtypeStruct((N, D), jnp.float32),
    mesh=plsc.VectorSubcoreMesh(core_axis_name="c", subcore_axis_name="s"),
    scratch_types=[
        pltpu.VMEM((PER_W,), jnp.int32),
        [pltpu.VMEM((C, D), jnp.float32) for _ in range(NBUF)],
        [pltpu.SemaphoreType.DMA for _ in range(NBUF)],
        [pltpu.SemaphoreType.DMA for _ in range(NBUF)],
    ],
)(_sc_gather_body)


def _ln_body(x_ref, g_ref, b_ref, o_ref):
    x = x_ref[...]
    mean = jnp.mean(x, axis=1, keepdims=True)
    cent = x - mean
    var = jnp.mean(cent * cent, axis=1, keepdims=True)
    o_ref[...] = cent * lax.rsqrt(var + EPS) * g_ref[...] + b_ref[...]


_LN_ROWS = 16384


def _tc_layernorm(x, gamma, beta):
    return pl.pallas_call(
        _ln_body,
        grid=(N // _LN_ROWS,),
        in_specs=[
            pl.BlockSpec((_LN_ROWS, D), lambda i: (i, 0)),
            pl.BlockSpec((1, D), lambda i: (0, 0)),
            pl.BlockSpec((1, D), lambda i: (0, 0)),
        ],
        out_specs=pl.BlockSpec((_LN_ROWS, D), lambda i: (i, 0)),
        out_shape=jax.ShapeDtypeStruct((N, D), jnp.float32),
    )(x, gamma.reshape(1, D), beta.reshape(1, D))


def kernel(input_ids, table, gamma, beta):
    ids = input_ids.reshape(-1).astype(jnp.int32)
    rows = _sc_gather(ids, table)
    out = _tc_layernorm(rows, gamma, beta)
    return out.reshape(BATCH, SEQ, D)


# speedup vs baseline: 2.8933x; 1.0162x over previous
import functools

import jax
import jax.numpy as jnp
from jax import lax
from jax.experimental import pallas as pl
from jax.experimental.pallas import tpu as pltpu
from jax.experimental.pallas import tpu_sc as plsc

VOCAB = 100000
D = 128
BATCH = 4096
SEQ = 200
N = BATCH * SEQ
EPS = 1e-12

NC = 2
NS = 16
NW = NC * NS
C = 128
NBUF = 4

S = 4
N_S = N // S
_LN_ROWS = 8192


def _make_sc_gather(n_rows):
    per_w = n_rows // NW
    nchunk = per_w // C
    assert per_w % C == 0 and nchunk >= 4

    def body(ids_hbm, table_hbm, out_hbm, idx_all, bufs, sems_in, sems_out):
        wid = lax.axis_index("s") * NC + lax.axis_index("c")
        base = wid * per_w

        pltpu.sync_copy(ids_hbm.at[pl.ds(base, per_w)], idx_all)

        def fire_gather(g, s):
            pltpu.async_copy(
                table_hbm.at[idx_all.at[pl.ds(g * C, C)]], bufs[s], sems_in[s])

        def wait_gather(g, s):
            pltpu.make_async_copy(
                table_hbm.at[idx_all.at[pl.ds(g * C, C)]], bufs[s],
                sems_in[s]).wait()

        def fire_write(g, s):
            pltpu.async_copy(
                bufs[s], out_hbm.at[pl.ds(base + g * C, C)], sems_out[s])

        def wait_write(g, s):
            pltpu.make_async_copy(
                bufs[s], out_hbm.at[pl.ds(base + g * C, C)], sems_out[s]).wait()

        fire_gather(0, 0)
        fire_gather(1, 1)
        for g in (0, 1):
            s = g % NBUF
            wait_gather(g, s)
            fire_write(g, s)
            fire_gather(g + 2, (g + 2) % NBUF)

        main_n = ((nchunk - 4) // NBUF) * NBUF

        @pl.loop(2, 2 + main_n, step=NBUF)
        def _outer(g0):
            for b in range(NBUF):
                g = g0 + b
                s = (2 + b) % NBUF
                wait_gather(g, s)
                fire_write(g, s)
                wait_write(g - 2, (s + 2) % NBUF)
                fire_gather(g + 2, (s + 2) % NBUF)

        for g in range(2 + main_n, nchunk):
            s = g % NBUF
            wait_gather(g, s)
            fire_write(g, s)
            wait_write(g - 2, (g - 2) % NBUF)
            if g + 2 < nchunk:
                fire_gather(g + 2, (g + 2) % NBUF)
        for g in (nchunk - 2, nchunk - 1):
            wait_write(g, g % NBUF)

    return functools.partial(
        pl.kernel,
        out_type=jax.ShapeDtypeStruct((n_rows, D), jnp.float32),
        mesh=plsc.VectorSubcoreMesh(core_axis_name="c", subcore_axis_name="s"),
        compiler_params=pltpu.CompilerParams(needs_layout_passes=False),
        scratch_types=[
            pltpu.VMEM((per_w,), jnp.int32),
            [pltpu.VMEM((C, D), jnp.float32) for _ in range(NBUF)],
            [pltpu.SemaphoreType.DMA for _ in range(NBUF)],
            [pltpu.SemaphoreType.DMA for _ in range(NBUF)],
        ],
    )(body)


_sc_gather_slice = _make_sc_gather(N_S)


def _ln_math(x_ref, g_ref, b_ref, o_ref):
    x = x_ref[...]
    mean = jnp.mean(x, axis=1, keepdims=True)
    cent = x - mean
    var = jnp.mean(cent * cent, axis=1, keepdims=True)
    o_ref[...] = cent * lax.rsqrt(var + EPS) * g_ref[...] + b_ref[...]


def _ln_body(x_ref, g_ref, b_ref, o_ref):
    _ln_math(x_ref, g_ref, b_ref, o_ref)


def _ln_body_acc(x_ref, g_ref, b_ref, acc_ref, o_ref):
    del acc_ref
    _ln_math(x_ref, g_ref, b_ref, o_ref)


def _tc_layernorm_slice(i, rows, gamma, beta, acc):
    blocks = N_S // _LN_ROWS
    x_spec = pl.BlockSpec((_LN_ROWS, D), lambda j: (j, 0))
    gb_spec = pl.BlockSpec((1, D), lambda j: (0, 0))
    out_spec = pl.BlockSpec(
        (_LN_ROWS, D), lambda j, i=i: (i * blocks + j, 0))
    out_shape = jax.ShapeDtypeStruct((N, D), jnp.float32)
    g2, b2 = gamma.reshape(1, D), beta.reshape(1, D)
    if acc is None:
        return pl.pallas_call(
            _ln_body,
            grid=(blocks,),
            in_specs=[x_spec, gb_spec, gb_spec],
            out_specs=out_spec,
            out_shape=out_shape,
        )(rows, g2, b2)
    return pl.pallas_call(
        _ln_body_acc,
        grid=(blocks,),
        in_specs=[x_spec, gb_spec, gb_spec,
                  pl.BlockSpec(memory_space=pl.ANY)],
        out_specs=out_spec,
        out_shape=out_shape,
        input_output_aliases={3: 0},
    )(rows, g2, b2, acc)


def kernel(input_ids, table, gamma, beta):
    ids = input_ids.reshape(-1).astype(jnp.int32)
    acc = None
    for i in range(S):
        rows_i = _sc_gather_slice(ids[i * N_S:(i + 1) * N_S], table)
        acc = _tc_layernorm_slice(i, rows_i, gamma, beta, acc)
    return acc.reshape(BATCH, SEQ, D)
